# baseline (speedup 1.0000x reference)
import jax
import jax.numpy as jnp
from jax import lax
from jax.experimental import pallas as pl
from jax.experimental.pallas import tpu as pltpu, tpu_sc as plsc

EMBED_DIM = 768
NUM_CORES = 2
NUM_SUBCORES = 16
NUM_WORKERS = NUM_CORES * NUM_SUBCORES
CHUNK = 32


def _sc_body(rows_per_w, batch, table_hbm, out_hbm, tbuf0, tbuf1,
             sbuf0, sbuf1, gsem0, gsem1, ssem0, ssem1):
    sid = lax.axis_index("s")
    wid = sid * NUM_CORES + lax.axis_index("c")
    base = wid * rows_per_w
    n = rows_per_w // CHUNK
    half = batch // 2
    tbufs, sbufs = [tbuf0, tbuf1], [sbuf0, sbuf1]
    gsems, ssems = [gsem0, gsem1], [ssem0, ssem1]
    gathers = [[] for _ in range(n)]
    scatters = [[] for _ in range(n)]

    def start_gathers(j):
        src = table_hbm.at[pl.ds(base + j * CHUNK, CHUNK)]
        gathers[j].append(
            pltpu.async_copy(src, tbufs[j % 2], gsems[j % 2]))
        gathers[j].append(
            pltpu.async_copy(src, sbufs[j % 2].at[sid], gsems[j % 2]))

    start_gathers(0)
    for j in range(n):
        if j + 1 < n:
            for c in scatters[j - 1] if j >= 1 else ():
                c.wait()
            start_gathers(j + 1)
        for g in gathers[j]:
            g.wait()
        dst_rows = pl.ds(base + j * CHUNK, CHUNK)
        for b in range(half):
            scatters[j].append(pltpu.async_copy(
                sbufs[j % 2].at[sid], out_hbm.at[b, dst_rows],
                ssems[j % 2]))
        for b in range(half, batch):
            scatters[j].append(pltpu.async_copy(
                tbufs[j % 2], out_hbm.at[b, dst_rows], ssems[j % 2]))
    for c in scatters[n - 2] + scatters[n - 1]:
        c.wait()


def kernel(x, table):
    batch, seq = x.shape
    max_len, d = table.shape
    assert seq == max_len and d == EMBED_DIM
    rows_per_w = max_len // NUM_WORKERS

    mesh = plsc.VectorSubcoreMesh(core_axis_name="c", subcore_axis_name="s")
    run = pl.kernel(
        lambda *refs: _sc_body(rows_per_w, batch, *refs),
        out_type=jax.ShapeDtypeStruct((batch, seq, d), jnp.float32),
        mesh=mesh,
        scratch_types=[
            pltpu.VMEM((CHUNK, d), jnp.float32),
            pltpu.VMEM((CHUNK, d), jnp.float32),
            pltpu.VMEM_SHARED((NUM_SUBCORES, CHUNK, d), jnp.float32),
            pltpu.VMEM_SHARED((NUM_SUBCORES, CHUNK, d), jnp.float32),
            pltpu.SemaphoreType.DMA,
            pltpu.SemaphoreType.DMA,
            pltpu.SemaphoreType.DMA,
            pltpu.SemaphoreType.DMA,
        ],
    )
    return run(table)


# speedup vs baseline: 1.2253x; 1.2253x over previous
import jax
import jax.numpy as jnp
from jax import lax
from jax.experimental import pallas as pl
from jax.experimental.pallas import tpu as pltpu, tpu_sc as plsc

EMBED_DIM = 768
NUM_CORES = 2
NUM_SUBCORES = 16
NUM_WORKERS = NUM_CORES * NUM_SUBCORES
CHUNK = 64


def _sc_body(rows_per_w, batch, table_hbm, out_hbm, buf0, buf1, gsem0, gsem1,
             ssem0, ssem1):
    wid = lax.axis_index("s") * NUM_CORES + lax.axis_index("c")
    base = wid * rows_per_w
    n = rows_per_w // CHUNK
    bufs, gsems, ssems = [buf0, buf1], [gsem0, gsem1], [ssem0, ssem1]
    gathers = [None] * n
    scatters = [[] for _ in range(n)]

    def start_gather(j):
        gathers[j] = pltpu.async_copy(
            table_hbm.at[pl.ds(base + j * CHUNK, CHUNK)], bufs[j % 2],
            gsems[j % 2])

    start_gather(0)
    for j in range(n):
        if j + 1 < n:
            for c in scatters[j - 1] if j >= 1 else ():
                c.wait()
            start_gather(j + 1)
        gathers[j].wait()
        for b in range(batch):
            b_rot = lax.rem(b + wid, batch)
            scatters[j].append(pltpu.async_copy(
                bufs[j % 2],
                out_hbm.at[b_rot, pl.ds(base + j * CHUNK, CHUNK)],
                ssems[j % 2]))
    for c in scatters[n - 2] + scatters[n - 1]:
        c.wait()


def kernel(x, table):
    batch, seq = x.shape
    max_len, d = table.shape
    assert seq == max_len and d == EMBED_DIM
    rows_per_w = max_len // NUM_WORKERS

    mesh = plsc.VectorSubcoreMesh(core_axis_name="c", subcore_axis_name="s")
    run = pl.kernel(
        lambda *refs: _sc_body(rows_per_w, batch, *refs),
        out_type=jax.ShapeDtypeStruct((batch, seq, d), jnp.float32),
        mesh=mesh,
        scratch_types=[
            pltpu.VMEM((CHUNK, d), jnp.float32),
            pltpu.VMEM((CHUNK, d), jnp.float32),
            pltpu.SemaphoreType.DMA,
            pltpu.SemaphoreType.DMA,
            pltpu.SemaphoreType.DMA,
            pltpu.SemaphoreType.DMA,
        ],
    )
    return run(table)
